# baseline (speedup 1.0000x reference)
import functools

import jax
import jax.numpy as jnp
from jax import lax
from jax.experimental import pallas as pl
from jax.experimental.pallas import tpu as pltpu, tpu_sc as plsc

N = 10000
K = 32
ROWS_PER_W = 313
VREGS = N // 16
UNROLL = 5
TOP_BITS = 0x7F800000
PROBE_HI = 0x40466666
PROBE_LO = 0x401CCCCD


def _splat(v):
    return jnp.full((16,), v, jnp.int32)


def _sc_kernel_body(a_hbm, out_hbm, row_buf):
    nc = 2
    wid = lax.axis_index("s") * nc + lax.axis_index("c")

    def do_row(r):
        pltpu.sync_copy(a_hbm.at[r], row_buf)

        def count(mid):
            mid_f = lax.bitcast_convert_type(_splat(mid), jnp.float32)

            def cbody(j, acc):
                for u in range(UNROLL):
                    v = row_buf[pl.ds((j * UNROLL + u) * 16, 16)]
                    acc = acc + jnp.where(v >= mid_f, 1, 0)
                return acc

            acc = lax.fori_loop(0, VREGS // UNROLL, cbody,
                                jnp.zeros((16,), jnp.int32))
            cnt = acc[0]
            for i in range(1, 16):
                cnt = cnt + acc[i]
            return cnt

        def bstep(mid, lo, hi, cnt_lo):
            cnt = count(mid)
            take_hi = cnt >= K
            return (jnp.where(take_hi, mid, lo), jnp.where(take_hi, hi, mid),
                    jnp.where(take_hi, cnt, cnt_lo))

        lo, hi, cnt_lo = bstep(PROBE_HI, 0, TOP_BITS, N)
        lo, hi, cnt_lo = bstep(jnp.maximum(PROBE_LO, lo), lo, hi, cnt_lo)

        def iter_body(_, c):
            lo, hi, cnt_lo = c

            def active(c2):
                lo2, hi2, cl2 = c2
                mid = lo2 + lax.shift_right_logical(hi2 - lo2, 1)
                return bstep(mid, lo2, hi2, cl2)

            unsettled = jnp.logical_and(cnt_lo != K, hi - lo > 1)
            return lax.cond(unsettled, active, lambda c2: c2, (lo, hi, cnt_lo))

        lo, _, _ = lax.fori_loop(0, 31, iter_body, (lo, hi, cnt_lo))

        lo_f = lax.bitcast_convert_type(_splat(lo), jnp.float32)

        def mbody(j, carry):
            for u in range(UNROLL):
                sl = pl.ds((j * UNROLL + u) * 16, 16)
                v = row_buf[sl]
                row_buf[sl] = jnp.where(v >= lo_f, v, 0.0)
            return carry

        lax.fori_loop(0, VREGS // UNROLL, mbody, 0)
        pltpu.sync_copy(row_buf, out_hbm.at[r])

    def row_body(i, carry):
        r = wid * ROWS_PER_W + i

        @pl.when(r < N)
        def _():
            do_row(r)

        return carry

    lax.fori_loop(0, ROWS_PER_W, row_body, 0)


@jax.jit
def _sc_topk_mask(a):
    mesh = plsc.VectorSubcoreMesh(core_axis_name="c", subcore_axis_name="s")
    fn = functools.partial(
        pl.kernel,
        mesh=mesh,
        out_type=jax.ShapeDtypeStruct((N, N), jnp.float32),
        scratch_types=[pltpu.VMEM((N,), jnp.float32)],
    )(_sc_kernel_body)
    return fn(a)


def kernel(idx, A_param):
    del idx
    return _sc_topk_mask(A_param)


# speedup vs baseline: 2.1198x; 2.1198x over previous
import functools

import jax
import jax.numpy as jnp
from jax import lax
from jax.experimental import pallas as pl

N = 10000
K = 32
BLOCK_ROWS = 200
BITS_STEPS = 31
TOP_BITS = 0x7F800000
PROBE_HI = 0x40466666
PROBE_LO = 0x401CCCCD


def _topk_mask_kernel(a_ref, o_ref):
    x = a_ref[...]

    lo0 = jnp.zeros((BLOCK_ROWS, 1), jnp.int32)
    hi0 = jnp.full((BLOCK_ROWS, 1), TOP_BITS, jnp.int32)
    cnt0 = jnp.full((BLOCK_ROWS, 1), N, jnp.int32)

    def cond(carry):
        i, lo, hi, cnt_lo = carry
        settled = jnp.logical_or(cnt_lo == K, hi - lo <= 1)
        return jnp.logical_and(i < BITS_STEPS, jnp.logical_not(jnp.all(settled)))

    def step(mid, lo, hi, cnt_lo):
        mid_f = lax.bitcast_convert_type(mid, jnp.float32)
        cnt = jnp.sum((x >= mid_f).astype(jnp.float32), axis=1, keepdims=True).astype(jnp.int32)
        take_hi = cnt >= K
        return (jnp.where(take_hi, mid, lo), jnp.where(take_hi, hi, mid),
                jnp.where(take_hi, cnt, cnt_lo))

    def body(carry):
        i, lo, hi, cnt_lo = carry
        mid = lo + lax.shift_right_logical(hi - lo, 1)
        lo, hi, cnt_lo = step(mid, lo, hi, cnt_lo)
        return i + 1, lo, hi, cnt_lo

    probe_hi = jnp.full((BLOCK_ROWS, 1), PROBE_HI, jnp.int32)
    lo, hi, cnt_lo = step(probe_hi, lo0, hi0, cnt0)
    probe_lo = jnp.maximum(jnp.full((BLOCK_ROWS, 1), PROBE_LO, jnp.int32), lo)
    lo, hi, cnt_lo = step(probe_lo, lo, hi, cnt_lo)

    _, lo, _, _ = lax.while_loop(cond, body, (0, lo, hi, cnt_lo))
    lo_f = lax.bitcast_convert_type(lo, jnp.float32)
    o_ref[...] = jnp.where(x >= lo_f, x, 0.0)


@jax.jit
def _topk_mask(a):
    grid = (N // BLOCK_ROWS,)
    return pl.pallas_call(
        _topk_mask_kernel,
        grid=grid,
        in_specs=[pl.BlockSpec((BLOCK_ROWS, N), lambda i: (i, 0))],
        out_specs=pl.BlockSpec((BLOCK_ROWS, N), lambda i: (i, 0)),
        out_shape=jax.ShapeDtypeStruct((N, N), jnp.float32),
    )(a)


def kernel(idx, A_param):
    del idx
    return _topk_mask(A_param)
